# baseline (speedup 1.0000x reference)
import jax
import jax.numpy as jnp
from jax import lax
from jax.experimental import pallas as pl
from jax.experimental.pallas import tpu as pltpu
from jax.experimental.pallas import tpu_sc as plsc

N = 10000
E = 320000
D = 128
K = 16
ALPHA = 0.05
CEFF = (1.0 - ALPHA) / K

NC = 2
NS = 16
NW = NC * NS
EPW = E // NW
CH = 80
NCHUNK = EPW // CH
NB = 3
RPT = N // NS
_RCHUNKS = [(q * CH, CH) for q in range(RPT // CH)] + [(RPT - RPT % CH, RPT % CH)]
DG = 16

_MESH = plsc.VectorSubcoreMesh(core_axis_name="c", subcore_axis_name="s")
_F32 = jnp.float32
_SC_PARAMS = pltpu.CompilerParams(use_tc_tiling_on_sc=False)


def _deg_body(col_hbm, zn_hbm, ones_hbm, degs_hbm,
              deg_sh, col_v, ones_v, dbuf, sem):
    cid = lax.axis_index("c")
    sid = lax.axis_index("s")
    wid = sid * NC + cid
    base = sid * RPT
    pltpu.sync_copy(zn_hbm.at[pl.ds(0, RPT)], dbuf)
    pltpu.sync_copy(dbuf, deg_sh.at[pl.ds(base, RPT)])
    pltpu.sync_copy(ones_hbm, ones_v)
    pltpu.sync_copy(col_hbm.at[wid], col_v)
    plsc.subcore_barrier()

    def chunk(j, carry):
        pltpu.async_copy(ones_v, deg_sh.at[col_v.at[j]], sem, add=True).wait()
        return carry

    lax.fori_loop(0, NCHUNK, chunk, 0)
    plsc.subcore_barrier()
    pltpu.sync_copy(deg_sh.at[pl.ds(base, RPT)], dbuf)
    pltpu.sync_copy(dbuf, degs_hbm.at[cid, pl.ds(base, RPT)])


_deg_call = pl.kernel(
    _deg_body,
    out_type=jax.ShapeDtypeStruct((NC, N, DG), _F32),
    mesh=_MESH,
    scratch_types=[
        pltpu.VMEM_SHARED((N, DG), _F32),
        pltpu.VMEM((NCHUNK, CH), jnp.int32),
        pltpu.VMEM((CH, DG), _F32),
        pltpu.VMEM((RPT, DG), _F32),
        pltpu.SemaphoreType.DMA,
    ],
    compiler_params=_SC_PARAMS,
)


def _step_body(xs_hbm, pk_hbm, znd_hbm, parts_hbm,
               acc_sh, pk_v, ib0, ib1, ib2, buf0, buf1, buf2, gsem, ssem):
    cid = lax.axis_index("c")
    sid = lax.axis_index("s")
    wid = sid * NC + cid
    base = sid * RPT
    bufs = (buf0, buf1, buf2)
    ibs = (ib0, ib1, ib2)
    for off, ln in _RCHUNKS:
        src = pl.ds(base + off, ln)

        @pl.when(cid == 0)
        def _():
            pltpu.sync_copy(xs_hbm.at[src], acc_sh.at[src])

        @pl.when(cid != 0)
        def _():
            pltpu.sync_copy(znd_hbm.at[src], acc_sh.at[src])
    pltpu.sync_copy(pk_hbm.at[wid], pk_v)
    plsc.subcore_barrier()

    def unpack(j, s):
        for v in range(CH // 16):
            pv = pk_v[j, pl.ds(v * 16, 16)]
            ibs[s][0, pl.ds(v * 16, 16)] = lax.shift_right_logical(pv, 14)
            ibs[s][1, pl.ds(v * 16, 16)] = lax.bitwise_and(pv, 16383)

    def gather(j, s):
        pltpu.async_copy(xs_hbm.at[ibs[s].at[0]], bufs[s], gsem)
        del j

    def gather_wait(s):
        pltpu.make_async_copy(xs_hbm.at[pl.ds(0, CH)], bufs[s], gsem).wait()

    def scatter(j, s):
        pltpu.async_copy(bufs[s], acc_sh.at[ibs[s].at[1]], ssem, add=True)
        del j

    def scatter_wait(s):
        pltpu.make_async_copy(znd_hbm.at[pl.ds(0, CH)], bufs[s], ssem).wait()

    def turn(j, s, do_wait_prev, do_prefetch):
        gather_wait(s)
        scatter(j, s)
        if do_wait_prev:
            scatter_wait((s + 2) % NB)
        if do_prefetch:
            sn = (s + 2) % NB
            unpack(j + 2, sn)
            gather(j + 2, sn)

    unpack(0, 0)
    gather(0, 0)
    unpack(1, 1)
    gather(1, 1)
    turn(0, 0, False, True)

    def body(g, carry):
        j0 = 3 * g + 1
        turn(j0, 1, True, True)
        turn(j0 + 1, 2, True, True)
        turn(j0 + 2, 0, True, True)
        return carry

    lax.fori_loop(0, (NCHUNK - 5) // NB, body, 0)
    turn(NCHUNK - 4, 1, True, True)
    turn(NCHUNK - 3, 2, True, True)
    turn(NCHUNK - 2, 0, True, False)
    turn(NCHUNK - 1, 1, True, False)
    scatter_wait(1)
    plsc.subcore_barrier()
    for off, ln in _RCHUNKS:
        pltpu.sync_copy(acc_sh.at[pl.ds(base + off, ln)],
                        parts_hbm.at[cid, pl.ds(base + off, ln)])


_step_call = pl.kernel(
    _step_body,
    out_type=jax.ShapeDtypeStruct((NC, N, D), _F32),
    mesh=_MESH,
    scratch_types=[
        pltpu.VMEM_SHARED((N, D), _F32),
        pltpu.VMEM((NCHUNK, CH), jnp.int32),
        pltpu.VMEM((2, CH), jnp.int32),
        pltpu.VMEM((2, CH), jnp.int32),
        pltpu.VMEM((2, CH), jnp.int32),
        pltpu.VMEM((CH, D), _F32),
        pltpu.VMEM((CH, D), _F32),
        pltpu.VMEM((CH, D), _F32),
        pltpu.SemaphoreType.DMA,
        pltpu.SemaphoreType.DMA,
    ],
    compiler_params=_SC_PARAMS,
)


_RB = 1000


def _prep_body(deg0, deg1, x0, dis, dis2, xs):
    d = deg0[...][:, 0:1] + deg1[...][:, 0:1] + 1.0
    r = lax.rsqrt(d)
    dis[...] = r
    dis2[...] = r * r
    xs[...] = x0[...] * r


def _prep_call(deg0, deg1, x0):
    return pl.pallas_call(
        _prep_body,
        grid=(N // _RB,),
        in_specs=[
            pl.BlockSpec((_RB, DG), lambda i: (i, 0)),
            pl.BlockSpec((_RB, DG), lambda i: (i, 0)),
            pl.BlockSpec((_RB, D), lambda i: (i, 0)),
        ],
        out_specs=[
            pl.BlockSpec((_RB, 1), lambda i: (i, 0)),
            pl.BlockSpec((_RB, 1), lambda i: (i, 0)),
            pl.BlockSpec((_RB, D), lambda i: (i, 0)),
        ],
        out_shape=[
            jax.ShapeDtypeStruct((N, 1), _F32),
            jax.ShapeDtypeStruct((N, 1), _F32),
            jax.ShapeDtypeStruct((N, D), _F32),
        ],
    )(deg0, deg1, x0)


def _combine_body(a0, a1, dis2, xsn):
    xsn[...] = (a0[...] + a1[...]) * dis2[...]


def _combine_call(a0, a1, dis2):
    return pl.pallas_call(
        _combine_body,
        grid=(N // _RB,),
        in_specs=[
            pl.BlockSpec((_RB, D), lambda i: (i, 0)),
            pl.BlockSpec((_RB, D), lambda i: (i, 0)),
            pl.BlockSpec((_RB, 1), lambda i: (i, 0)),
        ],
        out_specs=pl.BlockSpec((_RB, D), lambda i: (i, 0)),
        out_shape=jax.ShapeDtypeStruct((N, D), _F32),
    )(a0, a1, dis2)


def _final_body(x0, dis, w_ref, b_ref, *rest):
    xsk, out = rest[:K], rest[K]
    acc = xsk[0][...]
    for k in range(1, K):
        acc = acc + xsk[k][...]
    h = ALPHA * x0[...] + (CEFF / dis[...]) * acc
    out[...] = (jnp.dot(h, w_ref[...], preferred_element_type=jnp.float32)
                + b_ref[...])


def _final_call(x0, dis, w, b2, xs_list):
    return pl.pallas_call(
        _final_body,
        grid=(N // _RB,),
        in_specs=[
            pl.BlockSpec((_RB, D), lambda i: (i, 0)),
            pl.BlockSpec((_RB, 1), lambda i: (i, 0)),
            pl.BlockSpec((D, D), lambda i: (0, 0)),
            pl.BlockSpec((1, D), lambda i: (0, 0)),
        ] + [pl.BlockSpec((_RB, D), lambda i: (i, 0)) for _ in range(K)],
        out_specs=pl.BlockSpec((_RB, D), lambda i: (i, 0)),
        out_shape=jax.ShapeDtypeStruct((N, D), _F32),
    )(x0, dis, w, b2, *xs_list)


def kernel(node_emb, edge_index, W, b):
    row = edge_index[0].reshape(NW, NCHUNK, CH)
    col = edge_index[1].reshape(NW, NCHUNK, CH)
    packed = jnp.left_shift(row, 14) | col
    zeros_nd = jnp.zeros((N, D), _F32)
    zeros_ng = jnp.zeros((N, DG), _F32)
    ones_ch = jnp.ones((CH, DG), _F32)

    degs = _deg_call(col, zeros_ng, ones_ch)
    dis, dis2, xs = _prep_call(degs[0], degs[1], node_emb)

    xs_list = []
    for _ in range(K):
        parts = _step_call(xs, packed, zeros_nd)
        xs = _combine_call(parts[0], parts[1], dis2)
        xs_list.append(xs)

    return _final_call(node_emb, dis, W, b.reshape(1, D), xs_list)


# speedup vs baseline: 1.0161x; 1.0161x over previous
import jax
import jax.numpy as jnp
from jax import lax
from jax.experimental import pallas as pl
from jax.experimental.pallas import tpu as pltpu
from jax.experimental.pallas import tpu_sc as plsc

N = 10000
E = 320000
D = 128
K = 16
ALPHA = 0.05
CEFF = (1.0 - ALPHA) / K

NC = 2
NS = 16
NW = NC * NS
EPW = E // NW
CH = 80
NCHUNK = EPW // CH
NB = 3
RPT = N // NS
_RCHUNKS = [(q * CH, CH) for q in range(RPT // CH)] + [(RPT - RPT % CH, RPT % CH)]
DG = 16

_MESH = plsc.VectorSubcoreMesh(core_axis_name="c", subcore_axis_name="s")
_F32 = jnp.float32
_SC_PARAMS = pltpu.CompilerParams(use_tc_tiling_on_sc=False)


def _deg_body(col_hbm, zn_hbm, ones_hbm, degs_hbm,
              deg_sh, col_v, ones_v, dbuf, sem):
    cid = lax.axis_index("c")
    sid = lax.axis_index("s")
    wid = sid * NC + cid
    base = sid * RPT
    pltpu.sync_copy(zn_hbm.at[pl.ds(0, RPT)], dbuf)
    pltpu.sync_copy(dbuf, deg_sh.at[pl.ds(base, RPT)])
    pltpu.sync_copy(ones_hbm, ones_v)
    pltpu.sync_copy(col_hbm.at[wid], col_v)
    plsc.subcore_barrier()

    def chunk(j, carry):
        pltpu.async_copy(ones_v, deg_sh.at[col_v.at[j]], sem, add=True)
        return carry

    lax.fori_loop(0, NCHUNK, chunk, 0)

    def drain(j, carry):
        pltpu.make_async_copy(zn_hbm.at[pl.ds(0, CH)],
                              ones_v, sem).wait()
        return carry

    lax.fori_loop(0, NCHUNK, drain, 0)
    plsc.subcore_barrier()
    pltpu.sync_copy(deg_sh.at[pl.ds(base, RPT)], dbuf)
    pltpu.sync_copy(dbuf, degs_hbm.at[cid, pl.ds(base, RPT)])


_deg_call = pl.kernel(
    _deg_body,
    out_type=jax.ShapeDtypeStruct((NC, N, DG), _F32),
    mesh=_MESH,
    scratch_types=[
        pltpu.VMEM_SHARED((N, DG), _F32),
        pltpu.VMEM((NCHUNK, CH), jnp.int32),
        pltpu.VMEM((CH, DG), _F32),
        pltpu.VMEM((RPT, DG), _F32),
        pltpu.SemaphoreType.DMA,
    ],
    compiler_params=_SC_PARAMS,
)


def _step_body(xs_hbm, pk_hbm, znd_hbm, parts_hbm,
               acc_sh, pk_v, ib0, ib1, ib2, buf0, buf1, buf2, gsem, ssem):
    cid = lax.axis_index("c")
    sid = lax.axis_index("s")
    wid = sid * NC + cid
    base = sid * RPT
    bufs = (buf0, buf1, buf2)
    ibs = (ib0, ib1, ib2)
    pltpu.sync_copy(znd_hbm.at[pl.ds(0, CH)], buf0)
    for off, ln in _RCHUNKS:
        pltpu.sync_copy(buf0.at[pl.ds(0, ln)], acc_sh.at[pl.ds(base + off, ln)])
    pltpu.sync_copy(pk_hbm.at[wid], pk_v)
    plsc.subcore_barrier()

    def unpack(j, s):
        for v in range(CH // 16):
            pv = pk_v[j, pl.ds(v * 16, 16)]
            ibs[s][0, pl.ds(v * 16, 16)] = lax.shift_right_logical(pv, 14)
            ibs[s][1, pl.ds(v * 16, 16)] = lax.bitwise_and(pv, 16383)

    def gather(j, s):
        pltpu.async_copy(xs_hbm.at[ibs[s].at[0]], bufs[s], gsem)
        del j

    def gather_wait(s):
        pltpu.make_async_copy(xs_hbm.at[pl.ds(0, CH)], bufs[s], gsem).wait()

    def scatter(j, s):
        pltpu.async_copy(bufs[s], acc_sh.at[ibs[s].at[1]], ssem, add=True)
        del j

    def scatter_wait(s):
        pltpu.make_async_copy(znd_hbm.at[pl.ds(0, CH)], bufs[s], ssem).wait()

    def turn(j, s, do_wait_prev, do_prefetch):
        gather_wait(s)
        scatter(j, s)
        if do_wait_prev:
            scatter_wait((s + 2) % NB)
        if do_prefetch:
            sn = (s + 2) % NB
            unpack(j + 2, sn)
            gather(j + 2, sn)

    unpack(0, 0)
    gather(0, 0)
    unpack(1, 1)
    gather(1, 1)
    turn(0, 0, False, True)

    def body(g, carry):
        j0 = 3 * g + 1
        turn(j0, 1, True, True)
        turn(j0 + 1, 2, True, True)
        turn(j0 + 2, 0, True, True)
        return carry

    lax.fori_loop(0, (NCHUNK - 5) // NB, body, 0)
    turn(NCHUNK - 4, 1, True, True)
    turn(NCHUNK - 3, 2, True, True)
    turn(NCHUNK - 2, 0, True, False)
    turn(NCHUNK - 1, 1, True, False)
    scatter_wait(1)
    plsc.subcore_barrier()
    for off, ln in _RCHUNKS:
        pltpu.sync_copy(acc_sh.at[pl.ds(base + off, ln)], buf0.at[pl.ds(0, ln)])
        pltpu.sync_copy(buf0.at[pl.ds(0, ln)],
                        parts_hbm.at[cid, pl.ds(base + off, ln)])


_step_call = pl.kernel(
    _step_body,
    out_type=jax.ShapeDtypeStruct((NC, N, D), _F32),
    mesh=_MESH,
    scratch_types=[
        pltpu.VMEM_SHARED((N, D), _F32),
        pltpu.VMEM((NCHUNK, CH), jnp.int32),
        pltpu.VMEM((2, CH), jnp.int32),
        pltpu.VMEM((2, CH), jnp.int32),
        pltpu.VMEM((2, CH), jnp.int32),
        pltpu.VMEM((CH, D), _F32),
        pltpu.VMEM((CH, D), _F32),
        pltpu.VMEM((CH, D), _F32),
        pltpu.SemaphoreType.DMA,
        pltpu.SemaphoreType.DMA,
    ],
    compiler_params=_SC_PARAMS,
)


_RB = 1000


def _prep_body(deg0, deg1, x0, dis, dis2, xs):
    d = deg0[...][:, 0:1] + deg1[...][:, 0:1] + 1.0
    r = lax.rsqrt(d)
    dis[...] = r
    dis2[...] = r * r
    xs[...] = x0[...] * r


def _prep_call(deg0, deg1, x0):
    return pl.pallas_call(
        _prep_body,
        grid=(N // _RB,),
        in_specs=[
            pl.BlockSpec((_RB, DG), lambda i: (i, 0)),
            pl.BlockSpec((_RB, DG), lambda i: (i, 0)),
            pl.BlockSpec((_RB, D), lambda i: (i, 0)),
        ],
        out_specs=[
            pl.BlockSpec((_RB, 1), lambda i: (i, 0)),
            pl.BlockSpec((_RB, 1), lambda i: (i, 0)),
            pl.BlockSpec((_RB, D), lambda i: (i, 0)),
        ],
        out_shape=[
            jax.ShapeDtypeStruct((N, 1), _F32),
            jax.ShapeDtypeStruct((N, 1), _F32),
            jax.ShapeDtypeStruct((N, D), _F32),
        ],
    )(deg0, deg1, x0)


def _combine_body(a0, a1, xsp, dis2, xsn):
    xsn[...] = (a0[...] + a1[...] + xsp[...]) * dis2[...]


def _combine_call(a0, a1, xsp, dis2):
    return pl.pallas_call(
        _combine_body,
        grid=(N // _RB,),
        in_specs=[
            pl.BlockSpec((_RB, D), lambda i: (i, 0)),
            pl.BlockSpec((_RB, D), lambda i: (i, 0)),
            pl.BlockSpec((_RB, D), lambda i: (i, 0)),
            pl.BlockSpec((_RB, 1), lambda i: (i, 0)),
        ],
        out_specs=pl.BlockSpec((_RB, D), lambda i: (i, 0)),
        out_shape=jax.ShapeDtypeStruct((N, D), _F32),
    )(a0, a1, xsp, dis2)


def _final_body(x0, dis, w_ref, b_ref, *rest):
    xsk, out = rest[:K], rest[K]
    acc = xsk[0][...]
    for k in range(1, K):
        acc = acc + xsk[k][...]
    h = ALPHA * x0[...] + (CEFF / dis[...]) * acc
    out[...] = (jnp.dot(h, w_ref[...], preferred_element_type=jnp.float32)
                + b_ref[...])


def _final_call(x0, dis, w, b2, xs_list):
    return pl.pallas_call(
        _final_body,
        grid=(N // _RB,),
        in_specs=[
            pl.BlockSpec((_RB, D), lambda i: (i, 0)),
            pl.BlockSpec((_RB, 1), lambda i: (i, 0)),
            pl.BlockSpec((D, D), lambda i: (0, 0)),
            pl.BlockSpec((1, D), lambda i: (0, 0)),
        ] + [pl.BlockSpec((_RB, D), lambda i: (i, 0)) for _ in range(K)],
        out_specs=pl.BlockSpec((_RB, D), lambda i: (i, 0)),
        out_shape=jax.ShapeDtypeStruct((N, D), _F32),
    )(x0, dis, w, b2, *xs_list)


def kernel(node_emb, edge_index, W, b):
    row = edge_index[0].reshape(NW, NCHUNK, CH)
    col = edge_index[1].reshape(NW, NCHUNK, CH)
    packed = jnp.left_shift(row, 14) | col
    zeros_nd = jnp.zeros((N, D), _F32)
    zeros_ng = jnp.zeros((N, DG), _F32)
    ones_ch = jnp.ones((CH, DG), _F32)

    degs = _deg_call(col, zeros_ng, ones_ch)
    dis, dis2, xs = _prep_call(degs[0], degs[1], node_emb)

    xs_list = []
    for _ in range(K):
        parts = _step_call(xs, packed, zeros_nd)
        xs = _combine_call(parts[0], parts[1], xs, dis2)
        xs_list.append(xs)

    return _final_call(node_emb, dis, W, b.reshape(1, D), xs_list)
